# baseline (speedup 1.0000x reference)
import jax
import jax.numpy as jnp
from jax.experimental import pallas as pl
from jax.experimental.pallas import tpu as pltpu

D_MODEL = 1024
N_EXPERTS = 8
TOP_K = 2
FFN = 4096
T = 2048
M = T * TOP_K

BM = 128
M_TILES = M // BM
W = M_TILES + N_EXPERTS - 1
BF = 512
N_F = FFN // BF

_CH = 512
_N_CH = T // _CH


def _plan_kernel(x_ref, rw_ref, pos_ref, wts_ref, wexp_ref, wtile_ref,
                 wrs_ref, wre_ref):
    x = x_ref[...]
    rw = rw_ref[...]
    logits = jax.lax.dot_general(
        x, rw, (((1,), (1,)), ((), ())), preferred_element_type=jnp.float32
    )
    m = jnp.max(logits, axis=1, keepdims=True)
    ex = jnp.exp(logits - m)
    probs = ex / jnp.sum(ex, axis=1, keepdims=True)
    idx = jax.lax.broadcasted_iota(jnp.int32, probs.shape, 1)
    big = jnp.int32(N_EXPERTS + 1)
    p1 = jnp.max(probs, axis=1, keepdims=True)
    i1 = jnp.min(jnp.where(probs == p1, idx, big), axis=1, keepdims=True)
    m1 = idx == i1
    probs2 = jnp.where(m1, -1.0, probs)
    p2 = jnp.max(probs2, axis=1, keepdims=True)
    i2 = jnp.min(jnp.where(probs2 == p2, idx, big), axis=1, keepdims=True)
    m2 = idx == i2
    denom = p1 + p2
    w1 = p1 / denom
    w2 = p2 / denom

    oh = m1.astype(jnp.float32) + m2.astype(jnp.float32)
    r = jax.lax.broadcasted_iota(jnp.int32, (_CH, _CH), 0)
    c = jax.lax.broadcasted_iota(jnp.int32, (_CH, _CH), 1)
    tri = (r > c).astype(jnp.float32)
    chunks = []
    carry = jnp.zeros((1, N_EXPERTS), jnp.float32)
    for ci in range(_N_CH):
        ohc = oh[ci * _CH:(ci + 1) * _CH, :]
        sc = jax.lax.dot_general(
            tri, ohc, (((1,), (0,)), ((), ())),
            preferred_element_type=jnp.float32) + carry
        chunks.append(sc)
        carry = carry + jnp.sum(ohc, axis=0, keepdims=True)
    s = jnp.concatenate(chunks, axis=0)
    counts = carry

    def _cumsum_lanes(row, exclusive):
        cols = []
        acc = jnp.zeros((1, 1), row.dtype)
        for e in range(N_EXPERTS):
            cur = acc + row[0:1, e:e + 1]
            cols.append(acc if exclusive else cur)
            acc = cur
        return jnp.concatenate(cols, axis=1)

    off = _cumsum_lanes(counts, True)
    off_end = off + counts

    base = off + s
    pos0 = jnp.sum(jnp.where(m1, base, 0.0), axis=1, keepdims=True)
    pos1 = jnp.sum(jnp.where(m2, base, 0.0), axis=1, keepdims=True)
    pos_ref[...] = jnp.concatenate([pos0, pos1], axis=1).astype(jnp.int32)
    wts_ref[...] = jnp.concatenate([w1, w2], axis=1)

    offi = off.astype(jnp.int32)
    endi = off_end.astype(jnp.int32)
    cnti = counts.astype(jnp.int32)
    start_t = offi // BM
    end_t = jnp.where(cnti > 0, (endi - 1) // BM, -1)
    tiles = jnp.where(cnti > 0, end_t - start_t + 1, 0)
    cum_in = _cumsum_lanes(tiles, False)
    cum_ex = _cumsum_lanes(tiles, True)
    total = cum_in[0:1, N_EXPERTS - 1:N_EXPERTS]

    wi = jax.lax.broadcasted_iota(jnp.int32, (1, W), 1)
    ew = jnp.zeros((1, W), jnp.int32)
    for e in range(N_EXPERTS):
        ew = ew + (cum_in[0:1, e:e + 1] <= wi).astype(jnp.int32)
    ew = jnp.minimum(ew, N_EXPERTS - 1)

    def sel(arr):
        out = jnp.zeros((1, W), jnp.int32)
        for e in range(N_EXPERTS):
            out = out + jnp.where(ew == e, arr[0:1, e:e + 1], 0)
        return out

    tile_w = sel(start_t) + (wi - sel(cum_ex))
    tile_w = jnp.clip(tile_w, 0, M_TILES - 1)
    rs = jnp.maximum(sel(offi), tile_w * BM)
    re = jnp.minimum(sel(endi), tile_w * BM + BM)
    re = jnp.where(wi < total, re, 0)
    wexp_ref[...] = ew
    wtile_ref[...] = tile_w
    wrs_ref[...] = rs
    wre_ref[...] = re


def _gather_kernel(p0_ref, p1_ref, x_ref, xs_ref):
    si = pl.program_id(0)
    sidx = si * BM * 4 + jax.lax.broadcasted_iota(jnp.int32, (BM * 4, 1), 0)
    p0 = p0_ref[0]
    p1 = p1_ref[0]
    perm = (p0 == sidx).astype(jnp.float32) + (p1 == sidx).astype(jnp.float32)
    xs_ref[...] = jax.lax.dot_general(
        perm, x_ref[...], (((1,), (0,)), ((), ())),
        preferred_element_type=jnp.float32)


def _group_mm_kernel(wexp_ref, wtile_ref, wrs_ref, wre_ref,
                     xs_ref, w1_ref, v1_ref, w2_ref, ys_ref):
    f = pl.program_id(0)
    w = pl.program_id(1)

    @pl.when((f == 0) & (w == 0))
    def _init():
        ys_ref[...] = jnp.zeros_like(ys_ref)

    rs = wrs_ref[0, w]
    re = wre_ref[0, w]
    st = wtile_ref[0, w]

    @pl.when(re > rs)
    def _work():
        xt = xs_ref[pl.ds(st * BM, BM), :]
        gate = jax.lax.dot_general(
            xt, w1_ref[0], (((1,), (1,)), ((), ())),
            preferred_element_type=jnp.float32)
        up = jax.lax.dot_general(
            xt, v1_ref[0], (((1,), (1,)), ((), ())),
            preferred_element_type=jnp.float32)
        act = gate * jax.lax.logistic(gate) * up
        gidx = st * BM + jax.lax.broadcasted_iota(jnp.int32, (BM, 1), 0)
        mask = (gidx >= rs) & (gidx < re)
        act = jnp.where(mask, act, 0.0)
        ys_ref[pl.ds(st * BM, BM), :] += jax.lax.dot_general(
            act, w2_ref[0], (((1,), (1,)), ((), ())),
            preferred_element_type=jnp.float32)


def _combine_kernel(pos_ref, wts_ref, ys_ref, out_ref):
    p = pos_ref[...]
    wt = wts_ref[...]
    bt = p.shape[0]
    sl = jax.lax.broadcasted_iota(jnp.int32, (bt, M), 1)
    a = jnp.where(sl == p[:, 0:1], wt[:, 0:1], 0.0) + jnp.where(
        sl == p[:, 1:2], wt[:, 1:2], 0.0)
    out_ref[...] = jax.lax.dot_general(
        a, ys_ref[...], (((1,), (0,)), ((), ())),
        preferred_element_type=jnp.float32)


def kernel(hidden_states, router_weight, ws, w2s):
    x = hidden_states.reshape(-1, D_MODEL)

    pos, wts, wexp, wtile, wrs, wre = pl.pallas_call(
        _plan_kernel,
        out_shape=(
            jax.ShapeDtypeStruct((T, TOP_K), jnp.int32),
            jax.ShapeDtypeStruct((T, TOP_K), jnp.float32),
            jax.ShapeDtypeStruct((1, W), jnp.int32),
            jax.ShapeDtypeStruct((1, W), jnp.int32),
            jax.ShapeDtypeStruct((1, W), jnp.int32),
            jax.ShapeDtypeStruct((1, W), jnp.int32),
        ),
    )(x, router_weight)

    posT = pos.T.reshape(TOP_K, 1, T)

    xs = pl.pallas_call(
        _gather_kernel,
        grid=(M // (BM * 4),),
        in_specs=[
            pl.BlockSpec((1, 1, T), lambda s: (0, 0, 0)),
            pl.BlockSpec((1, 1, T), lambda s: (1, 0, 0)),
            pl.BlockSpec((T, D_MODEL), lambda s: (0, 0)),
        ],
        out_specs=pl.BlockSpec((BM * 4, D_MODEL), lambda s: (s, 0)),
        out_shape=jax.ShapeDtypeStruct((M, D_MODEL), jnp.float32),
        compiler_params=pltpu.CompilerParams(
            dimension_semantics=("arbitrary",),
        ),
    )(posT, posT, x)

    ys = pl.pallas_call(
        _group_mm_kernel,
        grid_spec=pltpu.PrefetchScalarGridSpec(
            num_scalar_prefetch=4,
            grid=(N_F, W),
            in_specs=[
                pl.BlockSpec((M, D_MODEL), lambda f, w, se, st, rs, re: (0, 0)),
                pl.BlockSpec(
                    (1, BF, D_MODEL),
                    lambda f, w, se, st, rs, re: (se[0, w], f, 0)),
                pl.BlockSpec(
                    (1, BF, D_MODEL),
                    lambda f, w, se, st, rs, re: (se[0, w], N_F + f, 0)),
                pl.BlockSpec(
                    (1, D_MODEL, BF),
                    lambda f, w, se, st, rs, re: (se[0, w], 0, f)),
            ],
            out_specs=pl.BlockSpec(
                (M, D_MODEL), lambda f, w, se, st, rs, re: (0, 0)),
        ),
        out_shape=jax.ShapeDtypeStruct((M, D_MODEL), jnp.float32),
        compiler_params=pltpu.CompilerParams(
            dimension_semantics=("arbitrary", "arbitrary"),
        ),
    )(wexp, wtile, wrs, wre, xs, ws, ws, w2s)

    out = pl.pallas_call(
        _combine_kernel,
        grid=(T // 512,),
        in_specs=[
            pl.BlockSpec((512, TOP_K), lambda t: (t, 0)),
            pl.BlockSpec((512, TOP_K), lambda t: (t, 0)),
            pl.BlockSpec((M, D_MODEL), lambda t: (0, 0)),
        ],
        out_specs=pl.BlockSpec((512, D_MODEL), lambda t: (t, 0)),
        out_shape=jax.ShapeDtypeStruct((T, D_MODEL), jnp.float32),
        compiler_params=pltpu.CompilerParams(
            dimension_semantics=("arbitrary",),
        ),
    )(pos, wts, ys)

    return out.reshape(hidden_states.shape)


# speedup vs baseline: 1.4583x; 1.4583x over previous
import jax
import jax.numpy as jnp
from jax.experimental import pallas as pl
from jax.experimental.pallas import tpu as pltpu

D_MODEL = 1024
N_EXPERTS = 8
TOP_K = 2
FFN = 4096
T = 2048
M = T * TOP_K

BM = 256
M_TILES = M // BM
W = M_TILES + N_EXPERTS - 1
BF = 512
N_F = FFN // BF

_CH = 512
_N_CH = T // _CH


def _plan_kernel(x_ref, rw_ref, pos_ref, wts_ref, wexp_ref, wtile_ref,
                 wrs_ref, wre_ref):
    x = x_ref[...]
    rw = rw_ref[...]
    logits = jax.lax.dot_general(
        x, rw, (((1,), (1,)), ((), ())), preferred_element_type=jnp.float32
    )
    m = jnp.max(logits, axis=1, keepdims=True)
    ex = jnp.exp(logits - m)
    probs = ex / jnp.sum(ex, axis=1, keepdims=True)
    idx = jax.lax.broadcasted_iota(jnp.int32, probs.shape, 1)
    big = jnp.int32(N_EXPERTS + 1)
    p1 = jnp.max(probs, axis=1, keepdims=True)
    i1 = jnp.min(jnp.where(probs == p1, idx, big), axis=1, keepdims=True)
    m1 = idx == i1
    probs2 = jnp.where(m1, -1.0, probs)
    p2 = jnp.max(probs2, axis=1, keepdims=True)
    i2 = jnp.min(jnp.where(probs2 == p2, idx, big), axis=1, keepdims=True)
    m2 = idx == i2
    denom = p1 + p2
    w1 = p1 / denom
    w2 = p2 / denom

    oh = m1.astype(jnp.float32) + m2.astype(jnp.float32)
    r = jax.lax.broadcasted_iota(jnp.int32, (_CH, _CH), 0)
    c = jax.lax.broadcasted_iota(jnp.int32, (_CH, _CH), 1)
    tri = (r > c).astype(jnp.float32)
    chunks = []
    carry = jnp.zeros((1, N_EXPERTS), jnp.float32)
    for ci in range(_N_CH):
        ohc = oh[ci * _CH:(ci + 1) * _CH, :]
        sc = jax.lax.dot_general(
            tri, ohc, (((1,), (0,)), ((), ())),
            preferred_element_type=jnp.float32) + carry
        chunks.append(sc)
        carry = carry + jnp.sum(ohc, axis=0, keepdims=True)
    s = jnp.concatenate(chunks, axis=0)
    counts = carry

    def _cumsum_lanes(row, exclusive):
        cols = []
        acc = jnp.zeros((1, 1), row.dtype)
        for e in range(N_EXPERTS):
            cur = acc + row[0:1, e:e + 1]
            cols.append(acc if exclusive else cur)
            acc = cur
        return jnp.concatenate(cols, axis=1)

    off = _cumsum_lanes(counts, True)
    off_end = off + counts

    base = off + s
    pos0 = jnp.sum(jnp.where(m1, base, 0.0), axis=1, keepdims=True)
    pos1 = jnp.sum(jnp.where(m2, base, 0.0), axis=1, keepdims=True)
    pos_ref[...] = jnp.concatenate([pos0, pos1], axis=1).astype(jnp.int32)
    wts_ref[...] = jnp.concatenate([w1, w2], axis=1)

    offi = off.astype(jnp.int32)
    endi = off_end.astype(jnp.int32)
    cnti = counts.astype(jnp.int32)
    start_t = offi // BM
    end_t = jnp.where(cnti > 0, (endi - 1) // BM, -1)
    tiles = jnp.where(cnti > 0, end_t - start_t + 1, 0)
    cum_in = _cumsum_lanes(tiles, False)
    cum_ex = _cumsum_lanes(tiles, True)
    total = cum_in[0:1, N_EXPERTS - 1:N_EXPERTS]

    wi = jax.lax.broadcasted_iota(jnp.int32, (1, W), 1)
    ew = jnp.zeros((1, W), jnp.int32)
    for e in range(N_EXPERTS):
        ew = ew + (cum_in[0:1, e:e + 1] <= wi).astype(jnp.int32)
    ew = jnp.minimum(ew, N_EXPERTS - 1)

    def sel(arr):
        out = jnp.zeros((1, W), jnp.int32)
        for e in range(N_EXPERTS):
            out = out + jnp.where(ew == e, arr[0:1, e:e + 1], 0)
        return out

    tile_w = sel(start_t) + (wi - sel(cum_ex))
    tile_w = jnp.clip(tile_w, 0, M_TILES - 1)
    rs = jnp.maximum(sel(offi), tile_w * BM)
    re = jnp.minimum(sel(endi), tile_w * BM + BM)
    re = jnp.where(wi < total, re, 0)
    wexp_ref[...] = ew
    wtile_ref[...] = tile_w
    wrs_ref[...] = rs
    wre_ref[...] = re


def _gather_kernel(p0_ref, p1_ref, x_ref, xs_ref):
    si = pl.program_id(0)
    sidx = si * BM * 4 + jax.lax.broadcasted_iota(jnp.int32, (BM * 4, 1), 0)
    p0 = p0_ref[0]
    p1 = p1_ref[0]
    perm = (p0 == sidx).astype(jnp.float32) + (p1 == sidx).astype(jnp.float32)
    xs_ref[...] = jax.lax.dot_general(
        perm, x_ref[...], (((1,), (0,)), ((), ())),
        preferred_element_type=jnp.float32)


def _group_mm_kernel(wexp_ref, wtile_ref, wrs_ref, wre_ref,
                     xs_ref, w1_ref, v1_ref, w2_ref, ys_ref):
    f = pl.program_id(0)
    w = pl.program_id(1)

    @pl.when((f == 0) & (w == 0))
    def _init():
        ys_ref[...] = jnp.zeros_like(ys_ref)

    rs = wrs_ref[0, w]
    re = wre_ref[0, w]
    st = wtile_ref[0, w]

    @pl.when(re > rs)
    def _work():
        xt = xs_ref[pl.ds(st * BM, BM), :]
        gate = jax.lax.dot_general(
            xt, w1_ref[0], (((1,), (1,)), ((), ())),
            preferred_element_type=jnp.float32)
        up = jax.lax.dot_general(
            xt, v1_ref[0], (((1,), (1,)), ((), ())),
            preferred_element_type=jnp.float32)
        act = gate * jax.lax.logistic(gate) * up
        gidx = st * BM + jax.lax.broadcasted_iota(jnp.int32, (BM, 1), 0)
        mask = (gidx >= rs) & (gidx < re)
        act = jnp.where(mask, act, 0.0)
        ys_ref[pl.ds(st * BM, BM), :] += jax.lax.dot_general(
            act, w2_ref[0], (((1,), (1,)), ((), ())),
            preferred_element_type=jnp.float32)


def _combine_kernel(pos_ref, wts_ref, ys_ref, out_ref):
    p = pos_ref[...]
    wt = wts_ref[...]
    bt = p.shape[0]
    sl = jax.lax.broadcasted_iota(jnp.int32, (bt, M), 1)
    a = jnp.where(sl == p[:, 0:1], wt[:, 0:1], 0.0) + jnp.where(
        sl == p[:, 1:2], wt[:, 1:2], 0.0)
    out_ref[...] = jax.lax.dot_general(
        a, ys_ref[...], (((1,), (0,)), ((), ())),
        preferred_element_type=jnp.float32)


def kernel(hidden_states, router_weight, ws, w2s):
    x = hidden_states.reshape(-1, D_MODEL)

    pos, wts, wexp, wtile, wrs, wre = pl.pallas_call(
        _plan_kernel,
        out_shape=(
            jax.ShapeDtypeStruct((T, TOP_K), jnp.int32),
            jax.ShapeDtypeStruct((T, TOP_K), jnp.float32),
            jax.ShapeDtypeStruct((1, W), jnp.int32),
            jax.ShapeDtypeStruct((1, W), jnp.int32),
            jax.ShapeDtypeStruct((1, W), jnp.int32),
            jax.ShapeDtypeStruct((1, W), jnp.int32),
        ),
    )(x, router_weight)

    posT = pos.T.reshape(TOP_K, 1, T)

    xs = pl.pallas_call(
        _gather_kernel,
        grid=(M // (BM * 4),),
        in_specs=[
            pl.BlockSpec((1, 1, T), lambda s: (0, 0, 0)),
            pl.BlockSpec((1, 1, T), lambda s: (1, 0, 0)),
            pl.BlockSpec((T, D_MODEL), lambda s: (0, 0)),
        ],
        out_specs=pl.BlockSpec((BM * 4, D_MODEL), lambda s: (s, 0)),
        out_shape=jax.ShapeDtypeStruct((M, D_MODEL), jnp.float32),
        compiler_params=pltpu.CompilerParams(
            dimension_semantics=("arbitrary",),
        ),
    )(posT, posT, x)

    ys = pl.pallas_call(
        _group_mm_kernel,
        grid_spec=pltpu.PrefetchScalarGridSpec(
            num_scalar_prefetch=4,
            grid=(N_F, W),
            in_specs=[
                pl.BlockSpec((M, D_MODEL), lambda f, w, se, st, rs, re: (0, 0)),
                pl.BlockSpec(
                    (1, BF, D_MODEL),
                    lambda f, w, se, st, rs, re: (se[0, w], f, 0)),
                pl.BlockSpec(
                    (1, BF, D_MODEL),
                    lambda f, w, se, st, rs, re: (se[0, w], N_F + f, 0)),
                pl.BlockSpec(
                    (1, D_MODEL, BF),
                    lambda f, w, se, st, rs, re: (se[0, w], 0, f)),
            ],
            out_specs=pl.BlockSpec(
                (M, D_MODEL), lambda f, w, se, st, rs, re: (0, 0)),
        ),
        out_shape=jax.ShapeDtypeStruct((M, D_MODEL), jnp.float32),
        compiler_params=pltpu.CompilerParams(
            dimension_semantics=("arbitrary", "arbitrary"),
        ),
    )(wexp, wtile, wrs, wre, xs, ws, ws, w2s)

    out = pl.pallas_call(
        _combine_kernel,
        grid=(T // 512,),
        in_specs=[
            pl.BlockSpec((512, TOP_K), lambda t: (t, 0)),
            pl.BlockSpec((512, TOP_K), lambda t: (t, 0)),
            pl.BlockSpec((M, D_MODEL), lambda t: (0, 0)),
        ],
        out_specs=pl.BlockSpec((512, D_MODEL), lambda t: (t, 0)),
        out_shape=jax.ShapeDtypeStruct((T, D_MODEL), jnp.float32),
        compiler_params=pltpu.CompilerParams(
            dimension_semantics=("arbitrary",),
        ),
    )(pos, wts, ys)

    return out.reshape(hidden_states.shape)


# speedup vs baseline: 1.6125x; 1.1057x over previous
import jax
import jax.numpy as jnp
from jax.experimental import pallas as pl
from jax.experimental.pallas import tpu as pltpu

D_MODEL = 1024
N_EXPERTS = 8
TOP_K = 2
FFN = 4096
T = 2048
M = T * TOP_K

BM = 512
M_TILES = M // BM
W = M_TILES + N_EXPERTS - 1
BF = 512
N_F = FFN // BF

_CH = 512
_N_CH = T // _CH


def _plan_kernel(x_ref, rw_ref, pos_ref, wts_ref, wexp_ref, wtile_ref,
                 wrs_ref, wre_ref):
    x = x_ref[...]
    rw = rw_ref[...]
    logits = jax.lax.dot_general(
        x, rw, (((1,), (1,)), ((), ())), preferred_element_type=jnp.float32
    )
    m = jnp.max(logits, axis=1, keepdims=True)
    ex = jnp.exp(logits - m)
    probs = ex / jnp.sum(ex, axis=1, keepdims=True)
    idx = jax.lax.broadcasted_iota(jnp.int32, probs.shape, 1)
    big = jnp.int32(N_EXPERTS + 1)
    p1 = jnp.max(probs, axis=1, keepdims=True)
    i1 = jnp.min(jnp.where(probs == p1, idx, big), axis=1, keepdims=True)
    m1 = idx == i1
    probs2 = jnp.where(m1, -1.0, probs)
    p2 = jnp.max(probs2, axis=1, keepdims=True)
    i2 = jnp.min(jnp.where(probs2 == p2, idx, big), axis=1, keepdims=True)
    m2 = idx == i2
    denom = p1 + p2
    w1 = p1 / denom
    w2 = p2 / denom

    oh = m1.astype(jnp.float32) + m2.astype(jnp.float32)
    r = jax.lax.broadcasted_iota(jnp.int32, (_CH, _CH), 0)
    c = jax.lax.broadcasted_iota(jnp.int32, (_CH, _CH), 1)
    tri = (r > c).astype(jnp.float32)
    chunks = []
    carry = jnp.zeros((1, N_EXPERTS), jnp.float32)
    for ci in range(_N_CH):
        ohc = oh[ci * _CH:(ci + 1) * _CH, :]
        sc = jax.lax.dot_general(
            tri, ohc, (((1,), (0,)), ((), ())),
            preferred_element_type=jnp.float32) + carry
        chunks.append(sc)
        carry = carry + jnp.sum(ohc, axis=0, keepdims=True)
    s = jnp.concatenate(chunks, axis=0)
    counts = carry

    def _cumsum_lanes(row, exclusive):
        cols = []
        acc = jnp.zeros((1, 1), row.dtype)
        for e in range(N_EXPERTS):
            cur = acc + row[0:1, e:e + 1]
            cols.append(acc if exclusive else cur)
            acc = cur
        return jnp.concatenate(cols, axis=1)

    off = _cumsum_lanes(counts, True)
    off_end = off + counts

    base = off + s
    pos0 = jnp.sum(jnp.where(m1, base, 0.0), axis=1, keepdims=True)
    pos1 = jnp.sum(jnp.where(m2, base, 0.0), axis=1, keepdims=True)
    pos_ref[...] = jnp.concatenate([pos0, pos1], axis=1).astype(jnp.int32)
    wts_ref[...] = jnp.concatenate([w1, w2], axis=1)

    offi = off.astype(jnp.int32)
    endi = off_end.astype(jnp.int32)
    cnti = counts.astype(jnp.int32)
    start_t = offi // BM
    end_t = jnp.where(cnti > 0, (endi - 1) // BM, -1)
    tiles = jnp.where(cnti > 0, end_t - start_t + 1, 0)
    cum_in = _cumsum_lanes(tiles, False)
    cum_ex = _cumsum_lanes(tiles, True)
    total = cum_in[0:1, N_EXPERTS - 1:N_EXPERTS]

    wi = jax.lax.broadcasted_iota(jnp.int32, (1, W), 1)
    ew = jnp.zeros((1, W), jnp.int32)
    for e in range(N_EXPERTS):
        ew = ew + (cum_in[0:1, e:e + 1] <= wi).astype(jnp.int32)
    ew = jnp.minimum(ew, N_EXPERTS - 1)

    def sel(arr):
        out = jnp.zeros((1, W), jnp.int32)
        for e in range(N_EXPERTS):
            out = out + jnp.where(ew == e, arr[0:1, e:e + 1], 0)
        return out

    tile_w = sel(start_t) + (wi - sel(cum_ex))
    tile_w = jnp.clip(tile_w, 0, M_TILES - 1)
    rs = jnp.maximum(sel(offi), tile_w * BM)
    re = jnp.minimum(sel(endi), tile_w * BM + BM)
    re = jnp.where(wi < total, re, 0)
    wexp_ref[...] = ew
    wtile_ref[...] = tile_w
    wrs_ref[...] = rs
    wre_ref[...] = re


def _gather_kernel(p0_ref, p1_ref, x_ref, xs_ref):
    si = pl.program_id(0)
    sidx = si * BM * 4 + jax.lax.broadcasted_iota(jnp.int32, (BM * 4, 1), 0)
    p0 = p0_ref[0]
    p1 = p1_ref[0]
    perm = (p0 == sidx).astype(jnp.float32) + (p1 == sidx).astype(jnp.float32)
    xs_ref[...] = jax.lax.dot_general(
        perm, x_ref[...], (((1,), (0,)), ((), ())),
        preferred_element_type=jnp.float32)


def _group_mm_kernel(wexp_ref, wtile_ref, wrs_ref, wre_ref,
                     xs_ref, w1_ref, v1_ref, w2_ref, ys_ref):
    f = pl.program_id(0)
    w = pl.program_id(1)

    @pl.when((f == 0) & (w == 0))
    def _init():
        ys_ref[...] = jnp.zeros_like(ys_ref)

    rs = wrs_ref[0, w]
    re = wre_ref[0, w]
    st = wtile_ref[0, w]

    @pl.when(re > rs)
    def _work():
        xt = xs_ref[pl.ds(st * BM, BM), :]
        gate = jax.lax.dot_general(
            xt, w1_ref[0], (((1,), (1,)), ((), ())),
            preferred_element_type=jnp.float32)
        up = jax.lax.dot_general(
            xt, v1_ref[0], (((1,), (1,)), ((), ())),
            preferred_element_type=jnp.float32)
        act = gate * jax.lax.logistic(gate) * up
        gidx = st * BM + jax.lax.broadcasted_iota(jnp.int32, (BM, 1), 0)
        mask = (gidx >= rs) & (gidx < re)
        act = jnp.where(mask, act, 0.0)
        ys_ref[pl.ds(st * BM, BM), :] += jax.lax.dot_general(
            act, w2_ref[0], (((1,), (1,)), ((), ())),
            preferred_element_type=jnp.float32)


def _combine_kernel(pos_ref, wts_ref, ys_ref, out_ref):
    p = pos_ref[...]
    wt = wts_ref[...]
    bt = p.shape[0]
    sl = jax.lax.broadcasted_iota(jnp.int32, (bt, M), 1)
    a = jnp.where(sl == p[:, 0:1], wt[:, 0:1], 0.0) + jnp.where(
        sl == p[:, 1:2], wt[:, 1:2], 0.0)
    out_ref[...] = jax.lax.dot_general(
        a, ys_ref[...], (((1,), (0,)), ((), ())),
        preferred_element_type=jnp.float32)


def kernel(hidden_states, router_weight, ws, w2s):
    x = hidden_states.reshape(-1, D_MODEL)

    pos, wts, wexp, wtile, wrs, wre = pl.pallas_call(
        _plan_kernel,
        out_shape=(
            jax.ShapeDtypeStruct((T, TOP_K), jnp.int32),
            jax.ShapeDtypeStruct((T, TOP_K), jnp.float32),
            jax.ShapeDtypeStruct((1, W), jnp.int32),
            jax.ShapeDtypeStruct((1, W), jnp.int32),
            jax.ShapeDtypeStruct((1, W), jnp.int32),
            jax.ShapeDtypeStruct((1, W), jnp.int32),
        ),
    )(x, router_weight)

    posT = pos.T.reshape(TOP_K, 1, T)

    xs = pl.pallas_call(
        _gather_kernel,
        grid=(M // (BM * 4),),
        in_specs=[
            pl.BlockSpec((1, 1, T), lambda s: (0, 0, 0)),
            pl.BlockSpec((1, 1, T), lambda s: (1, 0, 0)),
            pl.BlockSpec((T, D_MODEL), lambda s: (0, 0)),
        ],
        out_specs=pl.BlockSpec((BM * 4, D_MODEL), lambda s: (s, 0)),
        out_shape=jax.ShapeDtypeStruct((M, D_MODEL), jnp.float32),
        compiler_params=pltpu.CompilerParams(
            dimension_semantics=("arbitrary",),
        ),
    )(posT, posT, x)

    ys = pl.pallas_call(
        _group_mm_kernel,
        grid_spec=pltpu.PrefetchScalarGridSpec(
            num_scalar_prefetch=4,
            grid=(N_F, W),
            in_specs=[
                pl.BlockSpec((M, D_MODEL), lambda f, w, se, st, rs, re: (0, 0)),
                pl.BlockSpec(
                    (1, BF, D_MODEL),
                    lambda f, w, se, st, rs, re: (se[0, w], f, 0)),
                pl.BlockSpec(
                    (1, BF, D_MODEL),
                    lambda f, w, se, st, rs, re: (se[0, w], N_F + f, 0)),
                pl.BlockSpec(
                    (1, D_MODEL, BF),
                    lambda f, w, se, st, rs, re: (se[0, w], 0, f)),
            ],
            out_specs=pl.BlockSpec(
                (M, D_MODEL), lambda f, w, se, st, rs, re: (0, 0)),
        ),
        out_shape=jax.ShapeDtypeStruct((M, D_MODEL), jnp.float32),
        compiler_params=pltpu.CompilerParams(
            dimension_semantics=("arbitrary", "arbitrary"),
        ),
    )(wexp, wtile, wrs, wre, xs, ws, ws, w2s)

    out = pl.pallas_call(
        _combine_kernel,
        grid=(T // 512,),
        in_specs=[
            pl.BlockSpec((512, TOP_K), lambda t: (t, 0)),
            pl.BlockSpec((512, TOP_K), lambda t: (t, 0)),
            pl.BlockSpec((M, D_MODEL), lambda t: (0, 0)),
        ],
        out_specs=pl.BlockSpec((512, D_MODEL), lambda t: (t, 0)),
        out_shape=jax.ShapeDtypeStruct((T, D_MODEL), jnp.float32),
        compiler_params=pltpu.CompilerParams(
            dimension_semantics=("arbitrary",),
        ),
    )(pos, wts, ys)

    return out.reshape(hidden_states.shape)
